# Initial kernel scaffold; baseline (speedup 1.0000x reference)
import jax
import jax.numpy as jnp
from jax.experimental import pallas as pl


def kernel(rs, ds, regions_weight, days_weight):
    raise NotImplementedError("write your pallas kernel here")


# speedup vs baseline: 1.9947x; 1.9947x over previous
import functools

import jax
import jax.numpy as jnp
from jax import lax
from jax.experimental import pallas as pl
from jax.experimental.pallas import tpu as pltpu
from jax.experimental.pallas import tpu_sc as plsc

B = 16384
D = 64
NC = 2
NS = 16
NW = NC * NS
BW = B // NW
NCHUNK = 4
CHUNK = BW // NCHUNK
GROUPS = BW // 16


def _sc_body(rs_hbm, ds_hbm, rw_hbm, dw_hbm, out_hbm,
             rs_v, ds_v, r_rows, d_rows, out_v, sem):
    wid = lax.axis_index("s") * NC + lax.axis_index("c")
    base = wid * BW

    for k in range(NCHUNK):
        pltpu.sync_copy(rs_hbm.at[pl.ds(base + k * CHUNK, CHUNK)], rs_v.at[k])
        pltpu.sync_copy(ds_hbm.at[pl.ds(base + k * CHUNK, CHUNK)], ds_v.at[k])

    cps = []
    for k in range(NCHUNK):
        cps.append(pltpu.async_copy(
            rw_hbm.at[rs_v.at[k]],
            r_rows.at[pl.ds(k * CHUNK, CHUNK)], sem))
        cps.append(pltpu.async_copy(
            dw_hbm.at[ds_v.at[k]],
            d_rows.at[pl.ds(k * CHUNK, CHUNK)], sem))
    for cp in cps:
        cp.wait()

    def group_body(g, carry):
        rowv = g * 16 + lax.iota(jnp.int32, 16)
        acc0 = jnp.zeros((16,), jnp.float32)
        acc1 = jnp.zeros((16,), jnp.float32)
        for j in range(0, D, 2):
            jv0 = jnp.full((16,), j, jnp.int32)
            jv1 = jnp.full((16,), j + 1, jnp.int32)
            acc0 = acc0 + (plsc.load_gather(r_rows, [rowv, jv0]) *
                           plsc.load_gather(d_rows, [rowv, jv0]))
            acc1 = acc1 + (plsc.load_gather(r_rows, [rowv, jv1]) *
                           plsc.load_gather(d_rows, [rowv, jv1]))
        out_v[pl.ds(g * 16, 16)] = acc0 + acc1
        return carry

    lax.fori_loop(0, GROUPS, group_body, 0)

    pltpu.sync_copy(out_v, out_hbm.at[pl.ds(base, BW)])


@functools.partial(jax.jit, static_argnames=())
def _run(rs, ds, regions_weight, days_weight):
    mesh = plsc.VectorSubcoreMesh(core_axis_name="c", subcore_axis_name="s")
    f = functools.partial(
        pl.kernel,
        out_type=jax.ShapeDtypeStruct((B,), jnp.float32),
        mesh=mesh,
        scratch_types=[
            pltpu.VMEM((NCHUNK, CHUNK), jnp.int32),
            pltpu.VMEM((NCHUNK, CHUNK), jnp.int32),
            pltpu.VMEM((BW, D), jnp.float32),
            pltpu.VMEM((BW, D), jnp.float32),
            pltpu.VMEM((BW,), jnp.float32),
            pltpu.SemaphoreType.DMA,
        ],
        compiler_params=pltpu.CompilerParams(
            needs_layout_passes=False, use_tc_tiling_on_sc=False),
    )(_sc_body)
    return f(rs, ds, regions_weight, days_weight)


def kernel(rs, ds, regions_weight, days_weight):
    return _run(rs.astype(jnp.int32), ds.astype(jnp.int32),
                regions_weight, days_weight)
